# baseline (speedup 1.0000x reference)
import jax
import jax.numpy as jnp
from jax import lax
from jax.experimental import pallas as pl
from jax.experimental.pallas import tpu as pltpu
from jax.experimental.pallas import tpu_sc as plsc

B, H, S, D = 16, 8, 2048, 128
ROWS = B * H
NTILE = 32
ROWS_PER_TILE = ROWS // NTILE
CH = 512
NCH = S // CH
BR = 4


def _sc_body(src, dst, bufA, bufB, sems):
    c = lax.axis_index("c")
    s = lax.axis_index("s")
    base = (c * 16 + s) * ROWS_PER_TILE
    bufs = (bufA, bufB)

    def chunk_slice(i):
        row = base + i // NCH
        off = (i % NCH) * CH
        return (row, pl.ds(off, CH), slice(None))

    def in_copy(i, b):
        return pltpu.make_async_copy(src.at[chunk_slice(i)], bufs[b], sems.at[b])

    def out_copy(i, b):
        return pltpu.make_async_copy(bufs[b], dst.at[chunk_slice(i)], sems.at[2 + b])

    n = ROWS_PER_TILE * NCH
    in_copy(0, 0).start()
    for i in range(n):
        b = i % 2
        nb = 1 - b
        in_copy(i, b).wait()
        if i + 1 < n:
            if i >= 1:
                out_copy(i - 1, nb).wait()
            in_copy(i + 1, nb).start()
        out_copy(i, b).start()
    out_copy(n - 2, n % 2).wait()
    out_copy(n - 1, (n - 1) % 2).wait()


TC_NBUF = 16
TC_LOOK = 8


def _tc_body(v_ref, vo_ref, *rest):
    bufs = rest[:TC_NBUF]
    sems = rest[TC_NBUF]

    def in_copy(g, b):
        return pltpu.make_async_copy(v_ref.at[g], bufs[b], sems.at[b])

    def out_copy(g, b):
        return pltpu.make_async_copy(bufs[b], vo_ref.at[g], sems.at[TC_NBUF + b])

    total = ROWS
    for g in range(TC_LOOK):
        in_copy(g, g % TC_NBUF).start()
    for g in range(total):
        b = g % TC_NBUF
        in_copy(g, b).wait()
        out_copy(g, b).start()
        j = g + TC_LOOK
        if j < total:
            bj = j % TC_NBUF
            if j >= TC_NBUF:
                out_copy(j - TC_NBUF, bj).wait()
            in_copy(j, bj).start()
    for g in range(total - TC_NBUF, total):
        out_copy(g, g % TC_NBUF).wait()


def _sc_copy(x):
    fn = pl.kernel(
        _sc_body,
        out_type=jax.ShapeDtypeStruct((ROWS, S, D), jnp.float32),
        mesh=plsc.VectorSubcoreMesh(core_axis_name="c", subcore_axis_name="s"),
        scratch_types=[
            pltpu.MemorySpace.VMEM((CH, D), jnp.float32),
            pltpu.MemorySpace.VMEM((CH, D), jnp.float32),
            pltpu.SemaphoreType.DMA((4,)),
        ],
    )
    return fn(x)


def _tc_copy(x):
    return pl.pallas_call(
        _tc_body,
        in_specs=[pl.BlockSpec(memory_space=pl.ANY)],
        out_specs=pl.BlockSpec(memory_space=pl.ANY),
        out_shape=jax.ShapeDtypeStruct((ROWS, S, D), jnp.float32),
        scratch_shapes=[pltpu.VMEM((S, D), jnp.float32)] * TC_NBUF
        + [pltpu.SemaphoreType.DMA((2 * TC_NBUF,))],
    )(x)


def kernel(k_val, v_val, k_cache, v_cache):
    k2 = k_val.reshape(ROWS, S, D)
    v2 = v_val.reshape(ROWS, S, D)
    ko = _sc_copy(k2)
    vo = _tc_copy(v2)
    return ko.reshape(B, H, S, D), vo.reshape(B, H, S, D)


# speedup vs baseline: 1.0041x; 1.0041x over previous
import jax
import jax.numpy as jnp
from jax import lax
from jax.experimental import pallas as pl
from jax.experimental.pallas import tpu as pltpu
from jax.experimental.pallas import tpu_sc as plsc

B, H, S, D = 16, 8, 2048, 128
ROWS = B * H
NTILE = 32
ROWS_PER_TILE = ROWS // NTILE
CH = 512
NCH = S // CH
BR = 4


def _sc_body(src, dst, bufA, bufB, sems):
    c = lax.axis_index("c")
    s = lax.axis_index("s")
    base = (c * 16 + s) * ROWS_PER_TILE
    bufs = (bufA, bufB)

    def chunk_slice(i):
        row = base + i // NCH
        off = (i % NCH) * CH
        return (row, pl.ds(off, CH), slice(None))

    def in_copy(i, b):
        return pltpu.make_async_copy(src.at[chunk_slice(i)], bufs[b], sems.at[b])

    def out_copy(i, b):
        return pltpu.make_async_copy(bufs[b], dst.at[chunk_slice(i)], sems.at[2 + b])

    n = ROWS_PER_TILE * NCH
    in_copy(0, 0).start()
    for i in range(n):
        b = i % 2
        nb = 1 - b
        in_copy(i, b).wait()
        if i + 1 < n:
            if i >= 1:
                out_copy(i - 1, nb).wait()
            in_copy(i + 1, nb).start()
        out_copy(i, b).start()
    out_copy(n - 2, n % 2).wait()
    out_copy(n - 1, (n - 1) % 2).wait()


def _tc_body(v_ref, vo_ref):
    vo_ref[...] = v_ref[...]


def _sc_copy(x):
    fn = pl.kernel(
        _sc_body,
        out_type=jax.ShapeDtypeStruct((ROWS, S, D), jnp.float32),
        mesh=plsc.VectorSubcoreMesh(core_axis_name="c", subcore_axis_name="s"),
        scratch_types=[
            pltpu.MemorySpace.VMEM((CH, D), jnp.float32),
            pltpu.MemorySpace.VMEM((CH, D), jnp.float32),
            pltpu.SemaphoreType.DMA((4,)),
        ],
    )
    return fn(x)


def _tc_copy(x):
    spec = pl.BlockSpec((BR, S, D), lambda i: (i, 0, 0))
    return pl.pallas_call(
        _tc_body,
        grid=(ROWS // BR,),
        in_specs=[spec],
        out_specs=spec,
        out_shape=jax.ShapeDtypeStruct((ROWS, S, D), jnp.float32),
    )(x)


def kernel(k_val, v_val, k_cache, v_cache):
    k2 = k_val.reshape(ROWS, S, D)
    v2 = v_val.reshape(ROWS, S, D)
    ko = _sc_copy(k2)
    vo = _tc_copy(v2)
    return ko.reshape(B, H, S, D), vo.reshape(B, H, S, D)


# speedup vs baseline: 1.1524x; 1.1476x over previous
import jax
import jax.numpy as jnp
from jax.experimental import pallas as pl

BR = 4


def _copy_body(k_ref, v_ref, ko_ref, vo_ref):
    ko_ref[...] = k_ref[...]
    vo_ref[...] = v_ref[...]


def kernel(k_val, v_val, k_cache, v_cache):
    B, H, S, D = k_val.shape
    rows = B * H
    k2 = k_val.reshape(rows, S, D)
    v2 = v_val.reshape(rows, S, D)
    spec = pl.BlockSpec((BR, S, D), lambda i: (i, 0, 0))
    out = pl.pallas_call(
        _copy_body,
        grid=(rows // BR,),
        in_specs=[spec, spec],
        out_specs=[spec, spec],
        out_shape=[jax.ShapeDtypeStruct((rows, S, D), k_val.dtype)] * 2,
    )(k2, v2)
    return out[0].reshape(B, H, S, D), out[1].reshape(B, H, S, D)
